# baseline (speedup 1.0000x reference)
import jax
import jax.numpy as jnp
import numpy as np
from jax import lax
from jax.experimental import pallas as pl
from jax.experimental.pallas import tpu as pltpu
from jax.experimental.pallas import tpu_sc as plsc

ETA_C = 20
K_C = 128
MAX_ENT_C = 100000
BATCH_C = 4096
NC, NS, L = 2, 16, 16
NW = NC * NS
PW = BATCH_C // NW
NCH = K_C // L
NG = PW // L
NB = 4
TS = 17
_N_CORR = ETA_C * BATCH_C


def _row_tree_sum(tile):
    rows = [tile[pl.ds(l * TS, L)] for l in range(L)]
    while len(rows) > 1:
        rows = [rows[k] + rows[k + 1] for k in range(0, len(rows), 2)]
    return rows[0]


def _body(s_idx, p_idx, o_idx, keep, repl, ent, rel, out_inp, out_corr,
          sidx_v, pidx_v, oidx_v, ri_all, k_all, w_sp, ring,
          inp_v, corr_all, tile, tiles, sems):
    wid = lax.axis_index("s") * NC + lax.axis_index("c")
    base = wid * PW
    lane = lax.broadcasted_iota(jnp.int32, (L,), 0)
    col0 = lane * TS

    pltpu.sync_copy(s_idx.at[pl.ds(base, PW)], sidx_v)
    pltpu.sync_copy(o_idx.at[pl.ds(base, PW)], oidx_v)
    pltpu.sync_copy(p_idx.at[pl.ds(base, PW)], pidx_v)
    pltpu.sync_copy(repl.at[:, pl.ds(base, PW)], ri_all)
    pltpu.sync_copy(keep.at[:, pl.ds(base, PW)], k_all)

    def fire(t, b):
        pltpu.async_copy(ent.at[ri_all.at[t]], ring.at[b], sems.at[b])

    def gwait(t, b):
        pltpu.make_async_copy(ent.at[ri_all.at[t]], ring.at[b],
                              sems.at[b]).wait()

    fire(0, 0)
    fire(1, 1)
    cp = pltpu.async_copy(rel.at[pidx_v], ring.at[2], sems.at[2])
    co = pltpu.async_copy(ent.at[oidx_v], ring.at[3], sems.at[3])
    co.wait()
    cp.wait()

    def po_group(g, carry):
        del carry
        for l in range(L):
            i = g * L + l
            for c in range(NCH):
                sl = pl.ds(c * L, L)
                w_sp[PW + i, sl] = ring[2, i, sl] * ring[3, i, sl]
        return 0

    lax.fori_loop(0, NG, po_group, 0)

    cs = pltpu.async_copy(ent.at[sidx_v], ring.at[3], sems.at[3])
    cs.wait()

    def sp_group(g, carry):
        del carry
        for l in range(L):
            i = g * L + l
            acc0 = jnp.zeros((L,), jnp.float32)
            acc1 = jnp.zeros((L,), jnp.float32)
            for c in range(NCH):
                sl = pl.ds(c * L, L)
                s = ring[3, i, sl]
                w_sp[i, sl] = s * ring[2, i, sl]
                if c % 2 == 0:
                    acc0 = acc0 + w_sp[PW + i, sl] * s
                else:
                    acc1 = acc1 + w_sp[PW + i, sl] * s
            plsc.store_scatter(tile, [col0 + l], acc0 + acc1)
        inp_v[pl.ds(g * L, L)] = _row_tree_sum(tile)
        return 0

    lax.fori_loop(0, NG, sp_group, 0)
    fire(2, 2)
    fire(3, 3)
    pltpu.sync_copy(inp_v, out_inp.at[pl.ds(base, PW)])

    def score_chunk(t, bb):
        @plsc.parallel_loop(0, NG, unroll=2)
        def group(g):
            kvec = k_all[t, pl.ds(g * L, L)]
            for l in range(L):
                i = g * L + l
                row = i + (1 - kvec[l]) * PW
                a0 = jnp.zeros((L,), jnp.float32)
                a1 = jnp.zeros((L,), jnp.float32)
                for c in range(NCH):
                    sl = pl.ds(c * L, L)
                    if c % 2 == 0:
                        a0 = a0 + w_sp[row, sl] * ring[bb, i, sl]
                    else:
                        a1 = a1 + w_sp[row, sl] * ring[bb, i, sl]
                plsc.store_scatter(tiles, [col0 + (g * (L * TS) + l)], a0 + a1)

        def red(g, carry2):
            del carry2
            rows = [tiles[pl.ds(g * (L * TS) + l * TS, L)] for l in range(L)]
            while len(rows) > 1:
                rows = [rows[k] + rows[k + 1] for k in range(0, len(rows), 2)]
            corr_all[t, pl.ds(g * L, L)] = rows[0]
            return 0

        lax.fori_loop(0, NG, red, 0)

    def quartet(v, carry):
        del carry
        for bb in range(NB):
            t = NB * v + bb
            gwait(t, bb)
            score_chunk(t, bb)

            @pl.when(t < ETA_C - NB)
            def _():
                fire(t + NB, bb)

        return 0

    lax.fori_loop(0, ETA_C // NB, quartet, 0)
    pltpu.sync_copy(corr_all, out_corr.at[:, pl.ds(base, PW)])


_sc_call = pl.kernel(
    _body,
    out_type=(
        jax.ShapeDtypeStruct((BATCH_C,), jnp.float32),
        jax.ShapeDtypeStruct((ETA_C, BATCH_C), jnp.float32),
    ),
    mesh=plsc.VectorSubcoreMesh(core_axis_name="c", subcore_axis_name="s"),
    compiler_params=pltpu.CompilerParams(needs_layout_passes=False),
    scratch_types=[
        pltpu.VMEM((PW,), jnp.int32),
        pltpu.VMEM((PW,), jnp.int32),
        pltpu.VMEM((PW,), jnp.int32),
        pltpu.VMEM((ETA_C, PW), jnp.int32),
        pltpu.VMEM((ETA_C, PW), jnp.int32),
        pltpu.VMEM((2 * PW, K_C), jnp.float32),
        pltpu.VMEM((NB, PW, K_C), jnp.float32),
        pltpu.VMEM((PW,), jnp.float32),
        pltpu.VMEM((ETA_C, PW), jnp.float32),
        pltpu.VMEM((L * TS,), jnp.float32),
        pltpu.VMEM((NG * L * TS,), jnp.float32),
        pltpu.SemaphoreType.DMA((NB,)),
    ],
)


@jax.jit
def kernel(inputs, ent_emb, rel_emb):
    s_idx = inputs[:, 0]
    p_idx = inputs[:, 1]
    o_idx = inputs[:, 2]
    ckey = jax.random.key(42)
    ka, kb = jax.random.split(ckey)
    keep = jax.random.randint(
        ka, (_N_CORR,), 0, 2, dtype=jnp.int32).reshape(ETA_C, BATCH_C)
    repl = jax.random.randint(
        kb, (_N_CORR,), 0, MAX_ENT_C, dtype=jnp.int32).reshape(ETA_C, BATCH_C)
    inp_score, corr2 = _sc_call(
        s_idx, p_idx, o_idx, keep, repl, ent_emb, rel_emb)
    return (inp_score, corr2.reshape(_N_CORR))


# speedup vs baseline: 1.0238x; 1.0238x over previous
import jax
import jax.numpy as jnp
import numpy as np
from jax import lax
from jax.experimental import pallas as pl
from jax.experimental.pallas import tpu as pltpu
from jax.experimental.pallas import tpu_sc as plsc

ETA_C = 20
K_C = 128
MAX_ENT_C = 100000
BATCH_C = 4096
NC, NS, L = 2, 16, 16
NW = NC * NS
PW = BATCH_C // NW
NCH = K_C // L
NG = PW // L
NB = 4
TS = 17
NPK = K_C // (2 * L)
_FMT = plsc.PackFormat.INTERLEAVED
NPK = K_C // (2 * L)
_FMT = plsc.PackFormat.INTERLEAVED
_N_CORR = ETA_C * BATCH_C


def _row_tree_sum(tile):
    rows = [tile[pl.ds(l * TS, L)] for l in range(L)]
    while len(rows) > 1:
        rows = [rows[k] + rows[k + 1] for k in range(0, len(rows), 2)]
    return rows[0]


def _body(s_idx, p_idx, o_idx, keep, repl, ent, rel, out_inp, out_corr,
          sidx_v, pidx_v, oidx_v, ri_all, k_all, w_sp, ring,
          inp_v, corr_all, tile, tiles, sems):
    wid = lax.axis_index("s") * NC + lax.axis_index("c")
    base = wid * PW
    lane = lax.broadcasted_iota(jnp.int32, (L,), 0)
    col0 = lane * TS

    def pack(a, b):
        return plsc.bitcast(plsc.pack(a, b, format=_FMT), jnp.float32)

    def unpack(x):
        return plsc.unpack(plsc.bitcast(x, jnp.bfloat16), format=_FMT)

    def pack(a, b):
        return plsc.bitcast(plsc.pack(a, b, format=_FMT), jnp.float32)

    def unpack(x):
        return plsc.unpack(plsc.bitcast(x, jnp.bfloat16), format=_FMT)

    pltpu.sync_copy(s_idx.at[pl.ds(base, PW)], sidx_v)
    pltpu.sync_copy(o_idx.at[pl.ds(base, PW)], oidx_v)
    pltpu.sync_copy(p_idx.at[pl.ds(base, PW)], pidx_v)
    pltpu.sync_copy(repl.at[:, pl.ds(base, PW)], ri_all)
    pltpu.sync_copy(keep.at[:, pl.ds(base, PW)], k_all)

    def fire(t, b):
        pltpu.async_copy(ent.at[ri_all.at[t]], ring.at[b], sems.at[b])

    def gwait(t, b):
        pltpu.make_async_copy(ent.at[ri_all.at[t]], ring.at[b],
                              sems.at[b]).wait()

    fire(0, 0)
    fire(1, 1)
    cp = pltpu.async_copy(rel.at[pidx_v], ring.at[2], sems.at[2])
    co = pltpu.async_copy(ent.at[oidx_v], ring.at[3], sems.at[3])
    co.wait()
    cp.wait()

    @plsc.parallel_loop(0, NG, unroll=2)
    def po_group(g):
        for l in range(L):
            i = g * L + l
            for c in range(NPK):
                lo = ring[2, i, pl.ds(2 * c * L, L)] * ring[3, i, pl.ds(2 * c * L, L)]
                hi = (ring[2, i, pl.ds((2 * c + 1) * L, L)]
                      * ring[3, i, pl.ds((2 * c + 1) * L, L)])
                w_sp[PW + i, pl.ds(c * L, L)] = pack(lo, hi)

    cs = pltpu.async_copy(ent.at[sidx_v], ring.at[3], sems.at[3])
    cs.wait()

    @plsc.parallel_loop(0, NG, unroll=2)
    def sp_group(g):
        for l in range(L):
            i = g * L + l
            acc0 = jnp.zeros((L,), jnp.float32)
            acc1 = jnp.zeros((L,), jnp.float32)
            for c in range(NPK):
                blk = pl.ds(c * L, L)
                po_lo, po_hi = unpack(w_sp[PW + i, blk])
                s_lo = ring[3, i, pl.ds(2 * c * L, L)]
                s_hi = ring[3, i, pl.ds((2 * c + 1) * L, L)]
                sp_lo = s_lo * ring[2, i, pl.ds(2 * c * L, L)]
                sp_hi = s_hi * ring[2, i, pl.ds((2 * c + 1) * L, L)]
                w_sp[i, blk] = pack(sp_lo, sp_hi)
                acc0 = acc0 + po_lo * s_lo
                acc1 = acc1 + po_hi * s_hi
            plsc.store_scatter(tiles, [col0 + (g * (L * TS) + l)], acc0 + acc1)

    def inp_red(g, carry):
        del carry
        rows = [tiles[pl.ds(g * (L * TS) + l * TS, L)] for l in range(L)]
        while len(rows) > 1:
            rows = [rows[k] + rows[k + 1] for k in range(0, len(rows), 2)]
        inp_v[pl.ds(g * L, L)] = rows[0]
        return 0

    lax.fori_loop(0, NG, inp_red, 0)
    fire(2, 2)
    fire(3, 3)
    pltpu.sync_copy(inp_v, out_inp.at[pl.ds(base, PW)])

    def score_chunk(t, bb):
        @plsc.parallel_loop(0, NG, unroll=2)
        def group(g):
            kvec = k_all[t, pl.ds(g * L, L)]
            for l in range(L):
                i = g * L + l
                row = i + (1 - kvec[l]) * PW
                a0 = jnp.zeros((L,), jnp.float32)
                a1 = jnp.zeros((L,), jnp.float32)
                for c in range(NPK):
                    w_lo, w_hi = unpack(w_sp[row, pl.ds(c * L, L)])
                    a0 = a0 + w_lo * ring[bb, i, pl.ds(2 * c * L, L)]
                    a1 = a1 + w_hi * ring[bb, i, pl.ds((2 * c + 1) * L, L)]
                plsc.store_scatter(tiles, [col0 + (g * (L * TS) + l)], a0 + a1)

        def red(g, carry2):
            del carry2
            rows = [tiles[pl.ds(g * (L * TS) + l * TS, L)] for l in range(L)]
            while len(rows) > 1:
                rows = [rows[k] + rows[k + 1] for k in range(0, len(rows), 2)]
            corr_all[t, pl.ds(g * L, L)] = rows[0]
            return 0

        lax.fori_loop(0, NG, red, 0)

    def quartet(v, carry):
        del carry
        for bb in range(NB):
            t = NB * v + bb
            gwait(t, bb)
            score_chunk(t, bb)

            @pl.when(t < ETA_C - NB)
            def _():
                fire(t + NB, bb)

        return 0

    lax.fori_loop(0, ETA_C // NB, quartet, 0)
    pltpu.sync_copy(corr_all, out_corr.at[:, pl.ds(base, PW)])


_sc_call = pl.kernel(
    _body,
    out_type=(
        jax.ShapeDtypeStruct((BATCH_C,), jnp.float32),
        jax.ShapeDtypeStruct((ETA_C, BATCH_C), jnp.float32),
    ),
    mesh=plsc.VectorSubcoreMesh(core_axis_name="c", subcore_axis_name="s"),
    compiler_params=pltpu.CompilerParams(needs_layout_passes=False),
    scratch_types=[
        pltpu.VMEM((PW,), jnp.int32),
        pltpu.VMEM((PW,), jnp.int32),
        pltpu.VMEM((PW,), jnp.int32),
        pltpu.VMEM((ETA_C, PW), jnp.int32),
        pltpu.VMEM((ETA_C, PW), jnp.int32),
        pltpu.VMEM((2 * PW, K_C // 2), jnp.float32),
        pltpu.VMEM((NB, PW, K_C), jnp.float32),
        pltpu.VMEM((PW,), jnp.float32),
        pltpu.VMEM((ETA_C, PW), jnp.float32),
        pltpu.VMEM((L * TS,), jnp.float32),
        pltpu.VMEM((NG * L * TS,), jnp.float32),
        pltpu.SemaphoreType.DMA((NB,)),
    ],
)


@jax.jit
def kernel(inputs, ent_emb, rel_emb):
    s_idx = inputs[:, 0]
    p_idx = inputs[:, 1]
    o_idx = inputs[:, 2]
    ckey = jax.random.key(42)
    ka, kb = jax.random.split(ckey)
    keep = jax.random.randint(
        ka, (_N_CORR,), 0, 2, dtype=jnp.int32).reshape(ETA_C, BATCH_C)
    repl = jax.random.randint(
        kb, (_N_CORR,), 0, MAX_ENT_C, dtype=jnp.int32).reshape(ETA_C, BATCH_C)
    inp_score, corr2 = _sc_call(
        s_idx, p_idx, o_idx, keep, repl, ent_emb, rel_emb)
    return (inp_score, corr2.reshape(_N_CORR))


# speedup vs baseline: 1.0424x; 1.0182x over previous
import jax
import jax.numpy as jnp
import numpy as np
from jax import lax
from jax.experimental import pallas as pl
from jax.experimental.pallas import tpu as pltpu
from jax.experimental.pallas import tpu_sc as plsc

ETA_C = 20
K_C = 128
MAX_ENT_C = 100000
BATCH_C = 4096
NC, NS, L = 2, 16, 16
NW = NC * NS
PW = BATCH_C // NW
NCH = K_C // L
NG = PW // L
TS = 17
_N_CORR = ETA_C * BATCH_C


def _row_tree_sum(tile):
    rows = [tile[pl.ds(l * TS, L)] for l in range(L)]
    while len(rows) > 1:
        rows = [rows[k] + rows[k + 1] for k in range(0, len(rows), 2)]
    return rows[0]


def _body(s_idx, p_idx, o_idx, keep, repl, ent, rel, out_inp, out_corr,
          sidx_v, pidx_v, oidx_v, ri_all, k_all, w_buf, p_buf,
          r_a, r_b, r_c, r_d, inp_v, corr_all, t_p0, t_d0, t_p1, t_d1,
          sem_s, sem_o, sem_p, sem_a, sem_b, sem_c, sem_d):
    wid = lax.axis_index("s") * NC + lax.axis_index("c")
    base = wid * PW
    lane = lax.broadcasted_iota(jnp.int32, (L,), 0)
    col0 = lane * TS

    pltpu.sync_copy(s_idx.at[pl.ds(base, PW)], sidx_v)
    pltpu.sync_copy(o_idx.at[pl.ds(base, PW)], oidx_v)
    pltpu.sync_copy(p_idx.at[pl.ds(base, PW)], pidx_v)
    pltpu.sync_copy(repl.at[:, pl.ds(base, PW)], ri_all)
    pltpu.sync_copy(keep.at[:, pl.ds(base, PW)], k_all)
    cs = pltpu.async_copy(ent.at[sidx_v], w_buf.at[pl.ds(0, PW)], sem_s)
    co = pltpu.async_copy(ent.at[oidx_v], w_buf.at[pl.ds(PW, PW)], sem_o)
    cp = pltpu.async_copy(rel.at[pidx_v], p_buf, sem_p)

    def fire(t, r_buf, sem):
        pltpu.async_copy(ent.at[ri_all.at[t]], r_buf, sem)

    def gwait(t, r_buf, sem):
        pltpu.make_async_copy(ent.at[ri_all.at[t]], r_buf, sem).wait()

    fire(0, r_a, sem_a)
    fire(1, r_b, sem_b)
    fire(2, r_c, sem_c)
    fire(3, r_d, sem_d)

    cs.wait()
    co.wait()
    cp.wait()

    def orig_group(g, carry):
        del carry
        for l in range(L):
            i = g * L + l
            acc0 = jnp.zeros((L,), jnp.float32)
            acc1 = jnp.zeros((L,), jnp.float32)
            for c in range(NCH):
                sl = pl.ds(c * L, L)
                s = w_buf[i, sl]
                o = w_buf[PW + i, sl]
                p = p_buf[i, sl]
                sp = s * p
                po = p * o
                if c % 2 == 0:
                    acc0 = acc0 + sp * o
                else:
                    acc1 = acc1 + sp * o
                w_buf[i, sl] = sp - po
                w_buf[PW + i, sl] = po
            plsc.store_scatter(t_p0, [col0 + l], acc0 + acc1)
        inp_v[pl.ds(g * L, L)] = _row_tree_sum(t_p0)
        return 0

    lax.fori_loop(0, NG, orig_group, 0)
    pltpu.sync_copy(inp_v, out_inp.at[pl.ds(base, PW)])

    def score_pair(t0, rx, ry):
        t1 = t0 + 1

        def group(g, carry):
            del carry
            for l in range(L):
                i = g * L + l
                ap0 = jnp.zeros((L,), jnp.float32)
                ad0 = jnp.zeros((L,), jnp.float32)
                ap1 = jnp.zeros((L,), jnp.float32)
                ad1 = jnp.zeros((L,), jnp.float32)
                for c in range(NCH):
                    sl = pl.ds(c * L, L)
                    po = w_buf[PW + i, sl]
                    d = w_buf[i, sl]
                    r0 = rx[i, sl]
                    r1 = ry[i, sl]
                    ap0 = ap0 + po * r0
                    ad0 = ad0 + d * r0
                    ap1 = ap1 + po * r1
                    ad1 = ad1 + d * r1
                col = col0 + l
                plsc.store_scatter(t_p0, [col], ap0)
                plsc.store_scatter(t_d0, [col], ad0)
                plsc.store_scatter(t_p1, [col], ap1)
                plsc.store_scatter(t_d1, [col], ad1)
            gl = pl.ds(g * L, L)
            kf0 = k_all[t0, gl].astype(jnp.float32)
            kf1 = k_all[t1, gl].astype(jnp.float32)
            corr_all[t0, gl] = _row_tree_sum(t_p0) + kf0 * _row_tree_sum(t_d0)
            corr_all[t1, gl] = _row_tree_sum(t_p1) + kf1 * _row_tree_sum(t_d1)
            return 0

        lax.fori_loop(0, NG, group, 0)

    def quad(v, carry):
        del carry
        t = 4 * v
        gwait(t, r_a, sem_a)
        gwait(t + 1, r_b, sem_b)
        score_pair(t, r_a, r_b)
        fire(t + 4, r_a, sem_a)
        fire(t + 5, r_b, sem_b)
        gwait(t + 2, r_c, sem_c)
        gwait(t + 3, r_d, sem_d)
        score_pair(t + 2, r_c, r_d)
        fire(t + 6, r_c, sem_c)
        fire(t + 7, r_d, sem_d)
        return 0

    lax.fori_loop(0, ETA_C // 4 - 1, quad, 0)
    gwait(ETA_C - 4, r_a, sem_a)
    gwait(ETA_C - 3, r_b, sem_b)
    score_pair(ETA_C - 4, r_a, r_b)
    gwait(ETA_C - 2, r_c, sem_c)
    gwait(ETA_C - 1, r_d, sem_d)
    score_pair(ETA_C - 2, r_c, r_d)

    pltpu.sync_copy(corr_all, out_corr.at[:, pl.ds(base, PW)])


_sc_call = pl.kernel(
    _body,
    out_type=(
        jax.ShapeDtypeStruct((BATCH_C,), jnp.float32),
        jax.ShapeDtypeStruct((ETA_C, BATCH_C), jnp.float32),
    ),
    mesh=plsc.VectorSubcoreMesh(core_axis_name="c", subcore_axis_name="s"),
    compiler_params=pltpu.CompilerParams(needs_layout_passes=False),
    scratch_types=[
        pltpu.VMEM((PW,), jnp.int32),
        pltpu.VMEM((PW,), jnp.int32),
        pltpu.VMEM((PW,), jnp.int32),
        pltpu.VMEM((ETA_C, PW), jnp.int32),
        pltpu.VMEM((ETA_C, PW), jnp.int32),
        pltpu.VMEM((2 * PW, K_C), jnp.float32),
        pltpu.VMEM((PW, K_C), jnp.float32),
        pltpu.VMEM((PW, K_C), jnp.float32),
        pltpu.VMEM((PW, K_C), jnp.float32),
        pltpu.VMEM((PW, K_C), jnp.float32),
        pltpu.VMEM((PW, K_C), jnp.float32),
        pltpu.VMEM((PW,), jnp.float32),
        pltpu.VMEM((ETA_C, PW), jnp.float32),
        pltpu.VMEM((L * TS,), jnp.float32),
        pltpu.VMEM((L * TS,), jnp.float32),
        pltpu.VMEM((L * TS,), jnp.float32),
        pltpu.VMEM((L * TS,), jnp.float32),
        pltpu.SemaphoreType.DMA,
        pltpu.SemaphoreType.DMA,
        pltpu.SemaphoreType.DMA,
        pltpu.SemaphoreType.DMA,
        pltpu.SemaphoreType.DMA,
        pltpu.SemaphoreType.DMA,
        pltpu.SemaphoreType.DMA,
    ],
)


@jax.jit
def kernel(inputs, ent_emb, rel_emb):
    s_idx = inputs[:, 0]
    p_idx = inputs[:, 1]
    o_idx = inputs[:, 2]
    ckey = jax.random.key(42)
    ka, kb = jax.random.split(ckey)
    keep = jax.random.randint(
        ka, (_N_CORR,), 0, 2, dtype=jnp.int32).reshape(ETA_C, BATCH_C)
    repl = jax.random.randint(
        kb, (_N_CORR,), 0, MAX_ENT_C, dtype=jnp.int32).reshape(ETA_C, BATCH_C)
    inp_score, corr2 = _sc_call(
        s_idx, p_idx, o_idx, keep, repl, ent_emb, rel_emb)
    return (inp_score, corr2.reshape(_N_CORR))


# speedup vs baseline: 1.0922x; 1.0478x over previous
import jax
import jax.numpy as jnp
import numpy as np
from jax import lax
from jax.experimental import pallas as pl
from jax.experimental.pallas import tpu as pltpu
from jax.experimental.pallas import tpu_sc as plsc

ETA_C = 20
K_C = 128
MAX_ENT_C = 100000
BATCH_C = 4096
NC, NS, L = 2, 16, 16
NW = NC * NS
PW = BATCH_C // NW
NCH = K_C // L
NG = PW // L
TS = 17
_N_CORR = ETA_C * BATCH_C


def _row_tree_sum(tile):
    rows = [tile[pl.ds(l * TS, L)] for l in range(L)]
    while len(rows) > 1:
        rows = [rows[k] + rows[k + 1] for k in range(0, len(rows), 2)]
    return rows[0]


def _body(s_idx, p_idx, o_idx, keep, repl, ent, rel, out_inp, out_corr,
          sidx_v, pidx_v, oidx_v, ri_all, k_all, w_buf, p_buf,
          r_a, r_b, r_c, r_d, inp_v, corr_all, t_p0, t_d0, t_p1, t_d1,
          sem_s, sem_o, sem_p, sem_a, sem_b, sem_c, sem_d):
    wid = lax.axis_index("s") * NC + lax.axis_index("c")
    base = wid * PW
    lane = lax.broadcasted_iota(jnp.int32, (L,), 0)
    col0 = lane * TS

    pltpu.sync_copy(s_idx.at[pl.ds(base, PW)], sidx_v)
    pltpu.sync_copy(o_idx.at[pl.ds(base, PW)], oidx_v)
    pltpu.sync_copy(p_idx.at[pl.ds(base, PW)], pidx_v)
    pltpu.sync_copy(repl.at[:, pl.ds(base, PW)], ri_all)
    pltpu.sync_copy(keep.at[:, pl.ds(base, PW)], k_all)
    cs = pltpu.async_copy(ent.at[sidx_v], w_buf.at[pl.ds(0, PW)], sem_s)
    co = pltpu.async_copy(ent.at[oidx_v], w_buf.at[pl.ds(PW, PW)], sem_o)
    cp = pltpu.async_copy(rel.at[pidx_v], p_buf, sem_p)

    def fire(t, r_buf, sem):
        pltpu.async_copy(ent.at[ri_all.at[t]], r_buf, sem)

    def gwait(t, r_buf, sem):
        pltpu.make_async_copy(ent.at[ri_all.at[t]], r_buf, sem).wait()

    fire(0, r_a, sem_a)
    fire(1, r_b, sem_b)
    fire(2, r_c, sem_c)
    fire(3, r_d, sem_d)

    cs.wait()
    co.wait()
    cp.wait()

    def orig_group(g, carry):
        del carry
        for l in range(L):
            i = g * L + l
            acc0 = jnp.zeros((L,), jnp.float32)
            acc1 = jnp.zeros((L,), jnp.float32)
            for c in range(NCH):
                sl = pl.ds(c * L, L)
                s = w_buf[i, sl]
                o = w_buf[PW + i, sl]
                p = p_buf[i, sl]
                sp = s * p
                po = p * o
                if c % 2 == 0:
                    acc0 = acc0 + sp * o
                else:
                    acc1 = acc1 + sp * o
                w_buf[i, sl] = sp - po
                w_buf[PW + i, sl] = po
            plsc.store_scatter(t_p0, [col0 + l], acc0 + acc1)
        inp_v[pl.ds(g * L, L)] = _row_tree_sum(t_p0)
        return 0

    lax.fori_loop(0, NG, orig_group, 0)
    pltpu.sync_copy(inp_v, out_inp.at[pl.ds(base, PW)])

    def score_pair(t0, rx, ry):
        t1 = t0 + 1

        def group(g, carry):
            del carry
            for lp in range(L // 2):
                accs = []
                for l in (2 * lp, 2 * lp + 1):
                    i = g * L + l
                    ap0 = jnp.zeros((L,), jnp.float32)
                    ad0 = jnp.zeros((L,), jnp.float32)
                    ap1 = jnp.zeros((L,), jnp.float32)
                    ad1 = jnp.zeros((L,), jnp.float32)
                    for c in range(NCH):
                        sl = pl.ds(c * L, L)
                        po = w_buf[PW + i, sl]
                        d = w_buf[i, sl]
                        r0 = rx[i, sl]
                        r1 = ry[i, sl]
                        ap0 = ap0 + po * r0
                        ad0 = ad0 + d * r0
                        ap1 = ap1 + po * r1
                        ad1 = ad1 + d * r1
                    accs.append((l, ap0, ad0, ap1, ad1))
                for l, ap0, ad0, ap1, ad1 in accs:
                    col = col0 + l
                    plsc.store_scatter(t_p0, [col], ap0)
                    plsc.store_scatter(t_d0, [col], ad0)
                    plsc.store_scatter(t_p1, [col], ap1)
                    plsc.store_scatter(t_d1, [col], ad1)
            gl = pl.ds(g * L, L)
            kf0 = k_all[t0, gl].astype(jnp.float32)
            kf1 = k_all[t1, gl].astype(jnp.float32)
            corr_all[t0, gl] = _row_tree_sum(t_p0) + kf0 * _row_tree_sum(t_d0)
            corr_all[t1, gl] = _row_tree_sum(t_p1) + kf1 * _row_tree_sum(t_d1)
            return 0

        lax.fori_loop(0, NG, group, 0)

    def quad(v, carry):
        del carry
        t = 4 * v
        gwait(t, r_a, sem_a)
        gwait(t + 1, r_b, sem_b)
        score_pair(t, r_a, r_b)
        fire(t + 4, r_a, sem_a)
        fire(t + 5, r_b, sem_b)
        gwait(t + 2, r_c, sem_c)
        gwait(t + 3, r_d, sem_d)
        score_pair(t + 2, r_c, r_d)
        fire(t + 6, r_c, sem_c)
        fire(t + 7, r_d, sem_d)
        return 0

    lax.fori_loop(0, ETA_C // 4 - 1, quad, 0)
    gwait(ETA_C - 4, r_a, sem_a)
    gwait(ETA_C - 3, r_b, sem_b)
    score_pair(ETA_C - 4, r_a, r_b)
    gwait(ETA_C - 2, r_c, sem_c)
    gwait(ETA_C - 1, r_d, sem_d)
    score_pair(ETA_C - 2, r_c, r_d)

    pltpu.sync_copy(corr_all, out_corr.at[:, pl.ds(base, PW)])


_sc_call = pl.kernel(
    _body,
    out_type=(
        jax.ShapeDtypeStruct((BATCH_C,), jnp.float32),
        jax.ShapeDtypeStruct((ETA_C, BATCH_C), jnp.float32),
    ),
    mesh=plsc.VectorSubcoreMesh(core_axis_name="c", subcore_axis_name="s"),
    compiler_params=pltpu.CompilerParams(needs_layout_passes=False),
    scratch_types=[
        pltpu.VMEM((PW,), jnp.int32),
        pltpu.VMEM((PW,), jnp.int32),
        pltpu.VMEM((PW,), jnp.int32),
        pltpu.VMEM((ETA_C, PW), jnp.int32),
        pltpu.VMEM((ETA_C, PW), jnp.int32),
        pltpu.VMEM((2 * PW, K_C), jnp.float32),
        pltpu.VMEM((PW, K_C), jnp.float32),
        pltpu.VMEM((PW, K_C), jnp.float32),
        pltpu.VMEM((PW, K_C), jnp.float32),
        pltpu.VMEM((PW, K_C), jnp.float32),
        pltpu.VMEM((PW, K_C), jnp.float32),
        pltpu.VMEM((PW,), jnp.float32),
        pltpu.VMEM((ETA_C, PW), jnp.float32),
        pltpu.VMEM((L * TS,), jnp.float32),
        pltpu.VMEM((L * TS,), jnp.float32),
        pltpu.VMEM((L * TS,), jnp.float32),
        pltpu.VMEM((L * TS,), jnp.float32),
        pltpu.SemaphoreType.DMA,
        pltpu.SemaphoreType.DMA,
        pltpu.SemaphoreType.DMA,
        pltpu.SemaphoreType.DMA,
        pltpu.SemaphoreType.DMA,
        pltpu.SemaphoreType.DMA,
        pltpu.SemaphoreType.DMA,
    ],
)


@jax.jit
def kernel(inputs, ent_emb, rel_emb):
    s_idx = inputs[:, 0]
    p_idx = inputs[:, 1]
    o_idx = inputs[:, 2]
    ckey = jax.random.key(42)
    ka, kb = jax.random.split(ckey)
    keep = jax.random.randint(
        ka, (_N_CORR,), 0, 2, dtype=jnp.int32).reshape(ETA_C, BATCH_C)
    repl = jax.random.randint(
        kb, (_N_CORR,), 0, MAX_ENT_C, dtype=jnp.int32).reshape(ETA_C, BATCH_C)
    inp_score, corr2 = _sc_call(
        s_idx, p_idx, o_idx, keep, repl, ent_emb, rel_emb)
    return (inp_score, corr2.reshape(_N_CORR))


# speedup vs baseline: 1.1065x; 1.0130x over previous
import jax
import jax.numpy as jnp
import numpy as np
from jax import lax
from jax.experimental import pallas as pl
from jax.experimental.pallas import tpu as pltpu
from jax.experimental.pallas import tpu_sc as plsc

ETA_C = 20
K_C = 128
MAX_ENT_C = 100000
BATCH_C = 4096
NC, NS, L = 2, 16, 16
NW = NC * NS
PW = BATCH_C // NW
NCH = K_C // L
NG = PW // L
TS = 17
_N_CORR = ETA_C * BATCH_C


def _row_tree_sum(tile):
    rows = [tile[pl.ds(l * TS, L)] for l in range(L)]
    while len(rows) > 1:
        rows = [rows[k] + rows[k + 1] for k in range(0, len(rows), 2)]
    return rows[0]


def _body(s_idx, p_idx, o_idx, keep, repl, ent, rel, out_inp, out_corr,
          sidx_v, pidx_v, oidx_v, ri_all, k_all, w_buf, p_buf,
          r_a, r_b, r_c, r_d, inp_v, corr_all, t_p0, t_d0, t_p1, t_d1,
          sem_s, sem_o, sem_p, sem_a, sem_b, sem_c, sem_d):
    wid = lax.axis_index("s") * NC + lax.axis_index("c")
    base = wid * PW
    lane = lax.broadcasted_iota(jnp.int32, (L,), 0)
    col0 = lane * TS

    pltpu.sync_copy(s_idx.at[pl.ds(base, PW)], sidx_v)
    pltpu.sync_copy(o_idx.at[pl.ds(base, PW)], oidx_v)
    pltpu.sync_copy(p_idx.at[pl.ds(base, PW)], pidx_v)
    pltpu.sync_copy(repl.at[:, pl.ds(base, PW)], ri_all)
    pltpu.sync_copy(keep.at[:, pl.ds(base, PW)], k_all)
    cs = pltpu.async_copy(ent.at[sidx_v], w_buf.at[pl.ds(0, PW)], sem_s)
    co = pltpu.async_copy(ent.at[oidx_v], w_buf.at[pl.ds(PW, PW)], sem_o)
    cp = pltpu.async_copy(rel.at[pidx_v], p_buf, sem_p)

    def fire(t, r_buf, sem):
        pltpu.async_copy(ent.at[ri_all.at[t]], r_buf, sem)

    def gwait(t, r_buf, sem):
        pltpu.make_async_copy(ent.at[ri_all.at[t]], r_buf, sem).wait()

    fire(0, r_a, sem_a)
    fire(1, r_b, sem_b)
    fire(2, r_c, sem_c)
    fire(3, r_d, sem_d)

    cs.wait()
    co.wait()
    cp.wait()

    def orig_group(g, carry):
        del carry
        for l in range(L):
            i = g * L + l
            acc0 = jnp.zeros((L,), jnp.float32)
            acc1 = jnp.zeros((L,), jnp.float32)
            for c in range(NCH):
                sl = pl.ds(c * L, L)
                s = w_buf[i, sl]
                o = w_buf[PW + i, sl]
                p = p_buf[i, sl]
                sp = s * p
                po = p * o
                if c % 2 == 0:
                    acc0 = acc0 + sp * o
                else:
                    acc1 = acc1 + sp * o
                w_buf[i, sl] = sp - po
                w_buf[PW + i, sl] = po
            plsc.store_scatter(t_p0, [col0 + l], acc0 + acc1)
        inp_v[pl.ds(g * L, L)] = _row_tree_sum(t_p0)
        return 0

    lax.fori_loop(0, NG, orig_group, 0)
    pltpu.sync_copy(inp_v, out_inp.at[pl.ds(base, PW)])

    def score_pair(t0, rx, ry):
        t1 = t0 + 1

        def group(g, carry):
            del carry
            for lp in range(L // 4):
                accs = []
                for l in (4 * lp, 4 * lp + 1, 4 * lp + 2, 4 * lp + 3):
                    i = g * L + l
                    ap0 = jnp.zeros((L,), jnp.float32)
                    ad0 = jnp.zeros((L,), jnp.float32)
                    ap1 = jnp.zeros((L,), jnp.float32)
                    ad1 = jnp.zeros((L,), jnp.float32)
                    for c in range(NCH):
                        sl = pl.ds(c * L, L)
                        po = w_buf[PW + i, sl]
                        d = w_buf[i, sl]
                        r0 = rx[i, sl]
                        r1 = ry[i, sl]
                        ap0 = ap0 + po * r0
                        ad0 = ad0 + d * r0
                        ap1 = ap1 + po * r1
                        ad1 = ad1 + d * r1
                    accs.append((l, ap0, ad0, ap1, ad1))
                for l, ap0, ad0, ap1, ad1 in accs:
                    col = col0 + l
                    plsc.store_scatter(t_p0, [col], ap0)
                    plsc.store_scatter(t_d0, [col], ad0)
                    plsc.store_scatter(t_p1, [col], ap1)
                    plsc.store_scatter(t_d1, [col], ad1)
            gl = pl.ds(g * L, L)
            kf0 = k_all[t0, gl].astype(jnp.float32)
            kf1 = k_all[t1, gl].astype(jnp.float32)
            corr_all[t0, gl] = _row_tree_sum(t_p0) + kf0 * _row_tree_sum(t_d0)
            corr_all[t1, gl] = _row_tree_sum(t_p1) + kf1 * _row_tree_sum(t_d1)
            return 0

        lax.fori_loop(0, NG, group, 0)

    def quad(v, carry):
        del carry
        t = 4 * v
        gwait(t, r_a, sem_a)
        gwait(t + 1, r_b, sem_b)
        score_pair(t, r_a, r_b)
        fire(t + 4, r_a, sem_a)
        fire(t + 5, r_b, sem_b)
        gwait(t + 2, r_c, sem_c)
        gwait(t + 3, r_d, sem_d)
        score_pair(t + 2, r_c, r_d)
        fire(t + 6, r_c, sem_c)
        fire(t + 7, r_d, sem_d)
        return 0

    lax.fori_loop(0, ETA_C // 4 - 1, quad, 0)
    gwait(ETA_C - 4, r_a, sem_a)
    gwait(ETA_C - 3, r_b, sem_b)
    score_pair(ETA_C - 4, r_a, r_b)
    gwait(ETA_C - 2, r_c, sem_c)
    gwait(ETA_C - 1, r_d, sem_d)
    score_pair(ETA_C - 2, r_c, r_d)

    pltpu.sync_copy(corr_all, out_corr.at[:, pl.ds(base, PW)])


_sc_call = pl.kernel(
    _body,
    out_type=(
        jax.ShapeDtypeStruct((BATCH_C,), jnp.float32),
        jax.ShapeDtypeStruct((ETA_C, BATCH_C), jnp.float32),
    ),
    mesh=plsc.VectorSubcoreMesh(core_axis_name="c", subcore_axis_name="s"),
    compiler_params=pltpu.CompilerParams(needs_layout_passes=False),
    scratch_types=[
        pltpu.VMEM((PW,), jnp.int32),
        pltpu.VMEM((PW,), jnp.int32),
        pltpu.VMEM((PW,), jnp.int32),
        pltpu.VMEM((ETA_C, PW), jnp.int32),
        pltpu.VMEM((ETA_C, PW), jnp.int32),
        pltpu.VMEM((2 * PW, K_C), jnp.float32),
        pltpu.VMEM((PW, K_C), jnp.float32),
        pltpu.VMEM((PW, K_C), jnp.float32),
        pltpu.VMEM((PW, K_C), jnp.float32),
        pltpu.VMEM((PW, K_C), jnp.float32),
        pltpu.VMEM((PW, K_C), jnp.float32),
        pltpu.VMEM((PW,), jnp.float32),
        pltpu.VMEM((ETA_C, PW), jnp.float32),
        pltpu.VMEM((L * TS,), jnp.float32),
        pltpu.VMEM((L * TS,), jnp.float32),
        pltpu.VMEM((L * TS,), jnp.float32),
        pltpu.VMEM((L * TS,), jnp.float32),
        pltpu.SemaphoreType.DMA,
        pltpu.SemaphoreType.DMA,
        pltpu.SemaphoreType.DMA,
        pltpu.SemaphoreType.DMA,
        pltpu.SemaphoreType.DMA,
        pltpu.SemaphoreType.DMA,
        pltpu.SemaphoreType.DMA,
    ],
)


@jax.jit
def kernel(inputs, ent_emb, rel_emb):
    s_idx = inputs[:, 0]
    p_idx = inputs[:, 1]
    o_idx = inputs[:, 2]
    ckey = jax.random.key(42)
    ka, kb = jax.random.split(ckey)
    keep = jax.random.randint(
        ka, (_N_CORR,), 0, 2, dtype=jnp.int32).reshape(ETA_C, BATCH_C)
    repl = jax.random.randint(
        kb, (_N_CORR,), 0, MAX_ENT_C, dtype=jnp.int32).reshape(ETA_C, BATCH_C)
    inp_score, corr2 = _sc_call(
        s_idx, p_idx, o_idx, keep, repl, ent_emb, rel_emb)
    return (inp_score, corr2.reshape(_N_CORR))
